# Initial kernel scaffold; baseline (speedup 1.0000x reference)
import jax
import jax.numpy as jnp
from jax.experimental import pallas as pl


def kernel(input_decodings, spans, template):
    raise NotImplementedError("write your pallas kernel here")


# speedup vs baseline: 12.3516x; 12.3516x over previous
import functools

import jax
import jax.numpy as jnp
from jax import lax
from jax.experimental import pallas as pl
from jax.experimental.pallas import tpu as pltpu

_B = 8
_N = 8
_M = 256
_V = 1024
_T = 16
_C = 9


def _copy_template_kernel(w_ref, x_ref, out_ref):
    i = pl.program_id(0)
    m_iota = lax.broadcasted_iota(jnp.int32, (_M, 1), 0)
    lane0 = lax.broadcasted_iota(jnp.int32, (1, _V), 1) == 0

    xs = [x_ref[0, n] for n in range(_N)]

    res = jnp.zeros((_M, _V), dtype=jnp.float32)
    start = jnp.int32(0)
    for t in range(_T):
        out_t = w_ref[i, t, 1] * xs[0]
        for n in range(1, _N):
            out_t = out_t + w_ref[i, t, n + 1] * xs[n]
        out_t = out_t + jnp.where((m_iota == 0) & lane0, w_ref[i, t, 0], 0.0)

        rowmax = jnp.max(out_t, axis=1, keepdims=True)
        is_zero = out_t[:, 0:1] >= rowmax
        first_zero = jnp.min(jnp.where(is_zero, m_iota, _M))
        out_len = jnp.minimum(first_zero, _M - start)

        rolled = pltpu.roll(out_t, start, 0)
        valid = (m_iota >= start) & (m_iota < start + out_len)
        res = jnp.where(valid, rolled, res)
        start = start + out_len

    out_ref[0] = res


@jax.jit
def kernel(input_decodings, spans, template):
    coef_mask = (jnp.arange(_C)[None, None, :] <= spans[:, None, None])
    w = template * coef_mask.astype(template.dtype)

    return pl.pallas_call(
        _copy_template_kernel,
        grid=(_B,),
        in_specs=[
            pl.BlockSpec(memory_space=pltpu.SMEM),
            pl.BlockSpec((1, _N, _M, _V), lambda i: (i, 0, 0, 0)),
        ],
        out_specs=pl.BlockSpec((1, _M, _V), lambda i: (i, 0, 0)),
        out_shape=jax.ShapeDtypeStruct((_B, _M, _V), jnp.float32),
    )(w, input_decodings)


# speedup vs baseline: 57.2620x; 4.6360x over previous
import functools

import jax
import jax.numpy as jnp
from jax import lax
from jax.experimental import pallas as pl
from jax.experimental.pallas import tpu as pltpu

_B = 8
_N = 8
_M = 256
_V = 1024
_T = 16
_C = 9


def _copy_template_kernel(w_ref, x_ref, out_ref, start_ref, addcol_ref):
    i = pl.program_id(0)
    m_iota = lax.broadcasted_iota(jnp.int32, (_M, 1), 0)
    lane0 = lax.broadcasted_iota(jnp.int32, (1, _V), 1) == 0
    neg_inf = jnp.float32(-jnp.inf)

    start_ref[0] = jnp.int32(0)
    addcol_ref[...] = jnp.zeros((_M, 1), dtype=jnp.float32)
    out_ref[0] = jnp.zeros((_M, _V), dtype=jnp.float32)

    for t in range(_T):

        @pl.when(start_ref[0] < _M)
        def _(t=t):
            start = start_ref[0]
            w0 = w_ref[i, t, 0]

            out_t = w_ref[i, t, 1] * x_ref[0, 0]
            for n in range(1, _N):
                out_t = out_t + w_ref[i, t, n + 1] * x_ref[0, n]

            rowmax = jnp.max(out_t, axis=1, keepdims=True)
            is_zero = out_t[:, 0:1] >= rowmax
            row0 = out_t[0:1, :]
            c00 = jnp.max(jnp.where(lane0, row0, neg_inf))
            rm_rest = jnp.max(jnp.where(lane0, neg_inf, row0))

            fz_rest = jnp.min(
                jnp.where(is_zero & (m_iota > 0), m_iota, _M)
            )
            first_zero = jnp.where((c00 + w0) >= rm_rest, 0, fz_rest)
            out_len = jnp.minimum(first_zero, _M - start)

            @pl.when(out_len > 0)
            def _():
                valid = (m_iota >= start) & (m_iota < start + out_len)

                @pl.when(start == 0)
                def _():
                    out_ref[0] = jnp.where(valid, out_t, out_ref[0])

                @pl.when(start > 0)
                def _():
                    rolled = pltpu.roll(out_t, start, 0)
                    out_ref[0] = jnp.where(valid, rolled, out_ref[0])

                addcol_ref[...] = addcol_ref[...] + jnp.where(
                    m_iota == start, w0, 0.0
                )

            start_ref[0] = start + out_len

    out_ref[0] = out_ref[0] + jnp.where(lane0, addcol_ref[...], 0.0)


@jax.jit
def kernel(input_decodings, spans, template):
    coef_mask = (jnp.arange(_C)[None, None, :] <= spans[:, None, None])
    w = template * coef_mask.astype(template.dtype)

    return pl.pallas_call(
        _copy_template_kernel,
        grid=(_B,),
        in_specs=[
            pl.BlockSpec(memory_space=pltpu.SMEM),
            pl.BlockSpec((1, _N, _M, _V), lambda i: (i, 0, 0, 0)),
        ],
        out_specs=pl.BlockSpec((1, _M, _V), lambda i: (i, 0, 0)),
        out_shape=jax.ShapeDtypeStruct((_B, _M, _V), jnp.float32),
        scratch_shapes=[
            pltpu.SMEM((1,), jnp.int32),
            pltpu.VMEM((_M, 1), jnp.float32),
        ],
    )(w, input_decodings)


# speedup vs baseline: 57.5274x; 1.0046x over previous
import functools

import jax
import jax.numpy as jnp
from jax import lax
from jax.experimental import pallas as pl
from jax.experimental.pallas import tpu as pltpu

_B = 8
_N = 8
_M = 256
_V = 1024
_T = 16
_C = 9


def _copy_template_kernel(w_ref, x_ref, out_ref, start_ref, addcol_ref):
    i = pl.program_id(0)
    m_iota = lax.broadcasted_iota(jnp.int32, (_M, 1), 0)
    lane0 = lax.broadcasted_iota(jnp.int32, (1, _V), 1) == 0
    neg_inf = jnp.float32(-jnp.inf)

    start_ref[0] = jnp.int32(0)
    addcol_ref[...] = jnp.zeros((_M, 1), dtype=jnp.float32)
    out_ref[0] = jnp.zeros((_M, _V), dtype=jnp.float32)

    for t in range(_T):

        @pl.when(start_ref[0] < _M)
        def _(t=t):
            start = start_ref[0]
            w0 = w_ref[i, t, 0]

            out_t = w_ref[i, t, 1] * x_ref[0, 0]
            for n in range(1, _N):
                out_t = out_t + w_ref[i, t, n + 1] * x_ref[0, n]

            rowmax = jnp.max(out_t, axis=1, keepdims=True)
            is_zero = out_t[:, 0:1] >= rowmax
            row0 = out_t[0:1, :]
            c00 = jnp.max(jnp.where(lane0, row0, neg_inf))
            rm_rest = jnp.max(jnp.where(lane0, neg_inf, row0))

            fz_rest = jnp.min(
                jnp.where(is_zero & (m_iota > 0), m_iota, _M)
            )
            first_zero = jnp.where((c00 + w0) >= rm_rest, 0, fz_rest)
            out_len = jnp.minimum(first_zero, _M - start)

            @pl.when(out_len > 0)
            def _():
                valid = (m_iota >= start) & (m_iota < start + out_len)

                @pl.when(start == 0)
                def _():
                    out_ref[0] = jnp.where(valid, out_t, out_ref[0])

                @pl.when(start > 0)
                def _():
                    rolled = pltpu.roll(out_t, start, 0)
                    out_ref[0] = jnp.where(valid, rolled, out_ref[0])

                addcol_ref[...] = addcol_ref[...] + jnp.where(
                    m_iota == start, w0, 0.0
                )

            start_ref[0] = start + out_len

    out_ref[0] = out_ref[0] + jnp.where(lane0, addcol_ref[...], 0.0)


@jax.jit
def kernel(input_decodings, spans, template):
    coef_mask = (jnp.arange(_C)[None, None, :] <= spans[:, None, None])
    w = template * coef_mask.astype(template.dtype)

    return pl.pallas_call(
        _copy_template_kernel,
        grid=(_B,),
        in_specs=[
            pl.BlockSpec(memory_space=pltpu.SMEM),
            pl.BlockSpec((1, _N, _M, _V), lambda i: (i, 0, 0, 0)),
        ],
        out_specs=pl.BlockSpec((1, _M, _V), lambda i: (i, 0, 0)),
        out_shape=jax.ShapeDtypeStruct((_B, _M, _V), jnp.float32),
        scratch_shapes=[
            pltpu.SMEM((1,), jnp.int32),
            pltpu.VMEM((_M, 1), jnp.float32),
        ],
        compiler_params=pltpu.CompilerParams(
            dimension_semantics=("parallel",),
        ),
    )(w, input_decodings)


# speedup vs baseline: 61.9385x; 1.0767x over previous
import functools

import jax
import jax.numpy as jnp
from jax import lax
from jax.experimental import pallas as pl
from jax.experimental.pallas import tpu as pltpu

_B = 8
_N = 8
_M = 256
_V = 1024
_T = 16
_C = 9


def _start_fetch(x_ref, xbuf_ref, sem_ref, spans_ref, idx, slot):
    s = spans_ref[idx]
    for n in range(_N):

        @pl.when(n < s)
        def _(n=n):
            pltpu.make_async_copy(
                x_ref.at[idx, n], xbuf_ref.at[slot, n], sem_ref.at[slot, n]
            ).start()

        @pl.when(n >= s)
        def _(n=n):
            xbuf_ref[slot, n] = jnp.zeros((_M, _V), dtype=jnp.float32)


def _wait_fetch(x_ref, xbuf_ref, sem_ref, spans_ref, idx, slot):
    s = spans_ref[idx]
    for n in range(_N):

        @pl.when(n < s)
        def _(n=n):
            pltpu.make_async_copy(
                x_ref.at[idx, n], xbuf_ref.at[slot, n], sem_ref.at[slot, n]
            ).wait()


def _copy_template_kernel(
    w_ref, spans_ref, x_ref, out_ref, xbuf_ref, start_ref, addcol_ref, sem_ref
):
    i = pl.program_id(0)
    slot = lax.rem(i, 2)
    m_iota = lax.broadcasted_iota(jnp.int32, (_M, 1), 0)
    lane0 = lax.broadcasted_iota(jnp.int32, (1, _V), 1) == 0
    neg_inf = jnp.float32(-jnp.inf)

    @pl.when(i == 0)
    def _():
        _start_fetch(x_ref, xbuf_ref, sem_ref, spans_ref, 0, 0)

    @pl.when(i + 1 < _B)
    def _():
        _start_fetch(x_ref, xbuf_ref, sem_ref, spans_ref, i + 1, 1 - slot)

    _wait_fetch(x_ref, xbuf_ref, sem_ref, spans_ref, i, slot)

    start_ref[0] = jnp.int32(0)
    addcol_ref[...] = jnp.zeros((_M, 1), dtype=jnp.float32)
    out_ref[0] = jnp.zeros((_M, _V), dtype=jnp.float32)

    for t in range(_T):

        @pl.when(start_ref[0] < _M)
        def _(t=t):
            start = start_ref[0]
            w0 = w_ref[i, t, 0]

            out_t = w_ref[i, t, 1] * xbuf_ref[slot, 0]
            for n in range(1, _N):
                out_t = out_t + w_ref[i, t, n + 1] * xbuf_ref[slot, n]

            rowmax = jnp.max(out_t, axis=1, keepdims=True)
            is_zero = out_t[:, 0:1] >= rowmax
            row0 = out_t[0:1, :]
            c00 = jnp.max(jnp.where(lane0, row0, neg_inf))
            rm_rest = jnp.max(jnp.where(lane0, neg_inf, row0))

            fz_rest = jnp.min(
                jnp.where(is_zero & (m_iota > 0), m_iota, _M)
            )
            first_zero = jnp.where((c00 + w0) >= rm_rest, 0, fz_rest)
            out_len = jnp.minimum(first_zero, _M - start)

            @pl.when(out_len > 0)
            def _():
                valid = (m_iota >= start) & (m_iota < start + out_len)

                @pl.when(start == 0)
                def _():
                    out_ref[0] = jnp.where(valid, out_t, out_ref[0])

                @pl.when(start > 0)
                def _():
                    rolled = pltpu.roll(out_t, start, 0)
                    out_ref[0] = jnp.where(valid, rolled, out_ref[0])

                addcol_ref[...] = addcol_ref[...] + jnp.where(
                    m_iota == start, w0, 0.0
                )

            start_ref[0] = start + out_len

    out_ref[0] = out_ref[0] + jnp.where(lane0, addcol_ref[...], 0.0)


@jax.jit
def kernel(input_decodings, spans, template):
    coef_mask = (jnp.arange(_C)[None, None, :] <= spans[:, None, None])
    w = template * coef_mask.astype(template.dtype)
    spans_i32 = spans.astype(jnp.int32)

    return pl.pallas_call(
        _copy_template_kernel,
        grid=(_B,),
        in_specs=[
            pl.BlockSpec(memory_space=pltpu.SMEM),
            pl.BlockSpec(memory_space=pltpu.SMEM),
            pl.BlockSpec(memory_space=pltpu.MemorySpace.HBM),
        ],
        out_specs=pl.BlockSpec((1, _M, _V), lambda i: (i, 0, 0)),
        out_shape=jax.ShapeDtypeStruct((_B, _M, _V), jnp.float32),
        scratch_shapes=[
            pltpu.VMEM((2, _N, _M, _V), jnp.float32),
            pltpu.SMEM((1,), jnp.int32),
            pltpu.VMEM((_M, 1), jnp.float32),
            pltpu.SemaphoreType.DMA((2, _N)),
        ],
        compiler_params=pltpu.CompilerParams(
            dimension_semantics=("arbitrary",),
        ),
    )(w, spans_i32, input_decodings)
